# baseline (speedup 1.0000x reference)
import functools

import jax
import jax.numpy as jnp
from jax import lax
from jax.experimental import pallas as pl
from jax.experimental.pallas import tpu as pltpu
from jax.experimental.pallas import tpu_sc as plsc

_B = 1024
_K = 1024
_D = 64
_BB = 128
_KC = 256


def _vq_body(x_ref, w0_ref, b0_ref, w1_ref, b1_ref, w2_ref, b2_ref, cb_ref,
             u_ref, idx_ref):
    x = x_ref[...]
    h = jnp.maximum(
        jnp.dot(x, w0_ref[...], preferred_element_type=jnp.float32) + b0_ref[...], 0.0)
    h = jnp.maximum(
        jnp.dot(h, w1_ref[...], preferred_element_type=jnp.float32) + b1_ref[...], 0.0)
    u = jnp.dot(h, w2_ref[...], preferred_element_type=jnp.float32) + b2_ref[...]
    u_ref[...] = u

    u_aug = jnp.concatenate((u * -2.0, jnp.ones((_BB, 1), jnp.float32)), axis=1)

    def dist_step(i, carry):
        best_d, best_i = carry
        cb_c = cb_ref[pl.ds(i * _KC, _KC), :]
        c2 = jnp.sum(cb_c * cb_c, axis=1, keepdims=True)
        cb_aug = jnp.concatenate((cb_c, c2), axis=1)
        d = lax.dot_general(u_aug, cb_aug, (((1,), (1,)), ((), ())),
                            preferred_element_type=jnp.float32,
                            precision=lax.Precision.HIGHEST)
        dmin = jnp.min(d, axis=1, keepdims=True)
        iota = lax.broadcasted_iota(jnp.int32, d.shape, 1) + i * _KC
        imin = jnp.min(jnp.where(d == dmin, iota, _K), axis=1, keepdims=True)
        take = dmin < best_d
        return (jnp.where(take, dmin, best_d), jnp.where(take, imin, best_i))

    init = (jnp.full((_BB, 1), jnp.inf, jnp.float32),
            jnp.zeros((_BB, 1), jnp.int32))
    _, idx = lax.fori_loop(0, _K // _KC, dist_step, init)
    idx_ref[...] = idx


def _tc_stage(x, W0, b0, W1, b1, W2, b2, codebook):
    nsteps = _B // _BB
    row_block = lambda i: (i, 0)
    whole = lambda i: (0, 0)
    return pl.pallas_call(
        _vq_body,
        grid=(nsteps,),
        in_specs=[
            pl.BlockSpec((_BB, 640), row_block),
            pl.BlockSpec((640, 256), whole),
            pl.BlockSpec((1, 256), whole),
            pl.BlockSpec((256, 256), whole),
            pl.BlockSpec((1, 256), whole),
            pl.BlockSpec((256, _D), whole),
            pl.BlockSpec((1, _D), whole),
            pl.BlockSpec((_K, _D), whole),
        ],
        out_specs=(
            pl.BlockSpec((_BB, _D), row_block),
            pl.BlockSpec((_BB, 1), row_block),
        ),
        out_shape=(
            jax.ShapeDtypeStruct((_B, _D), jnp.float32),
            jax.ShapeDtypeStruct((_B, 1), jnp.int32),
        ),
        compiler_params=pltpu.CompilerParams(
            dimension_semantics=("arbitrary",),
        ),
    )(x, W0, b0[None, :], W1, b1[None, :], W2, b2[None, :], codebook)


_DP = 128


def _sc_gather(codebook_padded, idx):
    info = plsc.get_sparse_core_info()
    nc, ns = info.num_cores, info.num_subcores
    nw = nc * ns
    b_per_w = _B // nw
    mesh = plsc.VectorSubcoreMesh(core_axis_name="c", subcore_axis_name="s")

    @functools.partial(
        pl.kernel, mesh=mesh,
        out_type=jax.ShapeDtypeStruct((_B, _DP), jnp.float32),
        scratch_types=[
            pltpu.VMEM((b_per_w,), jnp.int32),
            pltpu.VMEM((b_per_w, _DP), jnp.float32),
            pltpu.SemaphoreType.DMA,
        ],
    )
    def gather_kernel(cb_hbm, idx_hbm, out_hbm, idx_v, rows_v, sem):
        wid = lax.axis_index("s") * nc + lax.axis_index("c")
        base = wid * b_per_w
        pltpu.sync_copy(idx_hbm.at[pl.ds(base, b_per_w)], idx_v)
        pltpu.async_copy(cb_hbm.at[idx_v], rows_v, sem).wait()
        pltpu.sync_copy(rows_v, out_hbm.at[pl.ds(base, b_per_w)])

    return gather_kernel(codebook_padded, idx)


def kernel(skills, language_operators, W0, b0, W1, b1, W2, b2, codebook):
    x = jnp.concatenate((skills, language_operators), axis=-1)
    u, idx2d = _tc_stage(x, W0, b0, W1, b1, W2, b2, codebook)
    cb_pad = jnp.pad(codebook, ((0, 0), (0, _DP - _D)))
    q = _sc_gather(cb_pad, idx2d.reshape(_B))[:, :_D]
    return (u, q)


# speedup vs baseline: 1.2959x; 1.2959x over previous
import functools

import jax
import jax.numpy as jnp
from jax import lax
from jax.experimental import pallas as pl
from jax.experimental.pallas import tpu as pltpu
from jax.experimental.pallas import tpu_sc as plsc

_B = 1024
_K = 1024
_D = 64
_BB = 128
_DP = 128


def _vq_body(s_ref, l_ref, w0_ref, b0_ref, w1_ref, b1_ref, w2_ref, b2_ref,
             cb_ref, u_ref, idx_ref):
    x = jnp.concatenate((s_ref[...], l_ref[...]), axis=1)
    h = jnp.maximum(
        jnp.dot(x, w0_ref[...], preferred_element_type=jnp.float32) + b0_ref[...], 0.0)
    h = jnp.maximum(
        jnp.dot(h, w1_ref[...], preferred_element_type=jnp.float32) + b1_ref[...], 0.0)
    u = jnp.dot(h, w2_ref[...], preferred_element_type=jnp.float32) + b2_ref[...]
    u_ref[...] = u

    u_aug = jnp.concatenate((u * -2.0, jnp.ones((_BB, 1), jnp.float32)), axis=1)
    cb = cb_ref[...]
    c2 = jnp.sum(cb * cb, axis=1, keepdims=True)
    cb_aug = jnp.concatenate((cb, c2), axis=1)
    d = lax.dot_general(u_aug, cb_aug, (((1,), (1,)), ((), ())),
                        preferred_element_type=jnp.float32,
                        precision=lax.Precision.HIGHEST)
    dmin = jnp.min(d, axis=1, keepdims=True)
    iota = lax.broadcasted_iota(jnp.int32, d.shape, 1)
    idx_ref[...] = jnp.min(jnp.where(d == dmin, iota, _K), axis=1,
                           keepdims=True)


def _tc_stage(skills, language_operators, W0, b0, W1, b1, W2, b2, codebook):
    nsteps = _B // _BB
    row_block = lambda i: (i, 0)
    whole = lambda i: (0, 0)
    return pl.pallas_call(
        _vq_body,
        grid=(nsteps,),
        in_specs=[
            pl.BlockSpec((_BB, 128), row_block),
            pl.BlockSpec((_BB, 512), row_block),
            pl.BlockSpec((640, 256), whole),
            pl.BlockSpec((1, 256), whole),
            pl.BlockSpec((256, 256), whole),
            pl.BlockSpec((1, 256), whole),
            pl.BlockSpec((256, _D), whole),
            pl.BlockSpec((1, _D), whole),
            pl.BlockSpec((_K, _D), whole),
        ],
        out_specs=(
            pl.BlockSpec((_BB, _D), row_block),
            pl.BlockSpec((_BB, 1), row_block),
        ),
        out_shape=(
            jax.ShapeDtypeStruct((_B, _D), jnp.float32),
            jax.ShapeDtypeStruct((_B, 1), jnp.int32),
        ),
        compiler_params=pltpu.CompilerParams(
            dimension_semantics=("arbitrary",),
        ),
    )(skills, language_operators, W0, b0[None, :], W1, b1[None, :],
      W2, b2[None, :], codebook)


def _sc_gather(codebook_padded, idx):
    info = plsc.get_sparse_core_info()
    nc, ns = info.num_cores, info.num_subcores
    nw = nc * ns
    b_per_w = _B // nw
    mesh = plsc.VectorSubcoreMesh(core_axis_name="c", subcore_axis_name="s")

    @functools.partial(
        pl.kernel, mesh=mesh,
        out_type=jax.ShapeDtypeStruct((_B, _DP), jnp.float32),
        scratch_types=[
            pltpu.VMEM((b_per_w,), jnp.int32),
            pltpu.VMEM((b_per_w, _DP), jnp.float32),
            pltpu.SemaphoreType.DMA,
        ],
    )
    def gather_kernel(cb_hbm, idx_hbm, out_hbm, idx_v, rows_v, sem):
        wid = lax.axis_index("s") * nc + lax.axis_index("c")
        base = wid * b_per_w
        pltpu.sync_copy(idx_hbm.at[pl.ds(base, b_per_w)], idx_v)
        pltpu.async_copy(cb_hbm.at[idx_v], rows_v, sem).wait()
        pltpu.sync_copy(rows_v, out_hbm.at[pl.ds(base, b_per_w)])

    return gather_kernel(codebook_padded, idx)


def kernel(skills, language_operators, W0, b0, W1, b1, W2, b2, codebook):
    u, idx2d = _tc_stage(skills, language_operators, W0, b0, W1, b1, W2, b2,
                         codebook)
    cb_pad = jnp.pad(codebook, ((0, 0), (0, _DP - _D)))
    q = _sc_gather(cb_pad, idx2d.reshape(_B))[:, :_D]
    return (u, q)


# speedup vs baseline: 1.3661x; 1.0542x over previous
import functools

import jax
import jax.numpy as jnp
from jax import lax
from jax.experimental import pallas as pl
from jax.experimental.pallas import tpu as pltpu
from jax.experimental.pallas import tpu_sc as plsc

_B = 1024
_K = 1024
_D = 64
_BB = 256
_DP = 128


def _vq_body(s_ref, l_ref, w0_ref, b0_ref, w1_ref, b1_ref, w2_ref, b2_ref,
             cb_ref, u_ref, idx_ref):
    x = jnp.concatenate((s_ref[...], l_ref[...]), axis=1)
    h = jnp.maximum(
        jnp.dot(x, w0_ref[...], preferred_element_type=jnp.float32) + b0_ref[...], 0.0)
    h = jnp.maximum(
        jnp.dot(h, w1_ref[...], preferred_element_type=jnp.float32) + b1_ref[...], 0.0)
    u = jnp.dot(h, w2_ref[...], preferred_element_type=jnp.float32) + b2_ref[...]
    u_ref[...] = u

    u_aug = jnp.concatenate((u * -2.0, jnp.ones((_BB, 1), jnp.float32)), axis=1)
    cb = cb_ref[...]
    c2 = jnp.sum(cb * cb, axis=1, keepdims=True)
    cb_aug = jnp.concatenate((cb, c2), axis=1)
    d = lax.dot_general(u_aug, cb_aug, (((1,), (1,)), ((), ())),
                        preferred_element_type=jnp.float32,
                        precision=lax.Precision.HIGHEST)
    dmin = jnp.min(d, axis=1, keepdims=True)
    iota = lax.broadcasted_iota(jnp.int32, d.shape, 1)
    idx_ref[...] = jnp.min(jnp.where(d == dmin, iota, _K), axis=1,
                           keepdims=True)


def _tc_stage(skills, language_operators, W0, b0, W1, b1, W2, b2, codebook):
    nsteps = _B // _BB
    row_block = lambda i: (i, 0)
    whole = lambda i: (0, 0)
    return pl.pallas_call(
        _vq_body,
        grid=(nsteps,),
        in_specs=[
            pl.BlockSpec((_BB, 128), row_block),
            pl.BlockSpec((_BB, 512), row_block),
            pl.BlockSpec((640, 256), whole),
            pl.BlockSpec((1, 256), whole),
            pl.BlockSpec((256, 256), whole),
            pl.BlockSpec((1, 256), whole),
            pl.BlockSpec((256, _D), whole),
            pl.BlockSpec((1, _D), whole),
            pl.BlockSpec((_K, _D), whole),
        ],
        out_specs=(
            pl.BlockSpec((_BB, _D), row_block),
            pl.BlockSpec((_BB, 1), row_block),
        ),
        out_shape=(
            jax.ShapeDtypeStruct((_B, _D), jnp.float32),
            jax.ShapeDtypeStruct((_B, 1), jnp.int32),
        ),
        compiler_params=pltpu.CompilerParams(
            dimension_semantics=("arbitrary",),
        ),
    )(skills, language_operators, W0, b0[None, :], W1, b1[None, :],
      W2, b2[None, :], codebook)


def _sc_gather(codebook_padded, idx):
    info = plsc.get_sparse_core_info()
    nc, ns = info.num_cores, info.num_subcores
    nw = nc * ns
    b_per_w = _B // nw
    mesh = plsc.VectorSubcoreMesh(core_axis_name="c", subcore_axis_name="s")

    @functools.partial(
        pl.kernel, mesh=mesh,
        out_type=jax.ShapeDtypeStruct((_B, _DP), jnp.float32),
        scratch_types=[
            pltpu.VMEM((b_per_w,), jnp.int32),
            pltpu.VMEM((b_per_w, _DP), jnp.float32),
            pltpu.SemaphoreType.DMA,
        ],
    )
    def gather_kernel(cb_hbm, idx_hbm, out_hbm, idx_v, rows_v, sem):
        wid = lax.axis_index("s") * nc + lax.axis_index("c")
        base = wid * b_per_w
        pltpu.sync_copy(idx_hbm.at[pl.ds(base, b_per_w)], idx_v)
        pltpu.async_copy(cb_hbm.at[idx_v], rows_v, sem).wait()
        pltpu.sync_copy(rows_v, out_hbm.at[pl.ds(base, b_per_w)])

    return gather_kernel(codebook_padded, idx)


def kernel(skills, language_operators, W0, b0, W1, b1, W2, b2, codebook):
    u, idx2d = _tc_stage(skills, language_operators, W0, b0, W1, b1, W2, b2,
                         codebook)
    cb_pad = jnp.pad(codebook, ((0, 0), (0, _DP - _D)))
    q = _sc_gather(cb_pad, idx2d.reshape(_B))[:, :_D]
    return (u, q)


# speedup vs baseline: 1.4055x; 1.0289x over previous
import functools

import jax
import jax.numpy as jnp
from jax import lax
from jax.experimental import pallas as pl
from jax.experimental.pallas import tpu as pltpu
from jax.experimental.pallas import tpu_sc as plsc

_B = 1024
_K = 1024
_D = 64
_BB = 512
_DP = 128


def _vq_body(s_ref, l_ref, w0_ref, b0_ref, w1_ref, b1_ref, w2_ref, b2_ref,
             cb_ref, u_ref, idx_ref):
    x = jnp.concatenate((s_ref[...], l_ref[...]), axis=1)
    h = jnp.maximum(
        jnp.dot(x, w0_ref[...], preferred_element_type=jnp.float32) + b0_ref[...], 0.0)
    h = jnp.maximum(
        jnp.dot(h, w1_ref[...], preferred_element_type=jnp.float32) + b1_ref[...], 0.0)
    u = jnp.dot(h, w2_ref[...], preferred_element_type=jnp.float32) + b2_ref[...]
    u_ref[...] = u

    u_aug = jnp.concatenate((u * -2.0, jnp.ones((_BB, 1), jnp.float32)), axis=1)
    cb = cb_ref[...]
    c2 = jnp.sum(cb * cb, axis=1, keepdims=True)
    cb_aug = jnp.concatenate((cb, c2), axis=1)
    d = lax.dot_general(u_aug, cb_aug, (((1,), (1,)), ((), ())),
                        preferred_element_type=jnp.float32,
                        precision=lax.Precision.HIGHEST)
    dmin = jnp.min(d, axis=1, keepdims=True)
    iota = lax.broadcasted_iota(jnp.int32, d.shape, 1)
    idx_ref[...] = jnp.min(jnp.where(d == dmin, iota, _K), axis=1,
                           keepdims=True)


def _tc_stage(skills, language_operators, W0, b0, W1, b1, W2, b2, codebook):
    nsteps = _B // _BB
    row_block = lambda i: (i, 0)
    whole = lambda i: (0, 0)
    return pl.pallas_call(
        _vq_body,
        grid=(nsteps,),
        in_specs=[
            pl.BlockSpec((_BB, 128), row_block),
            pl.BlockSpec((_BB, 512), row_block),
            pl.BlockSpec((640, 256), whole),
            pl.BlockSpec((1, 256), whole),
            pl.BlockSpec((256, 256), whole),
            pl.BlockSpec((1, 256), whole),
            pl.BlockSpec((256, _D), whole),
            pl.BlockSpec((1, _D), whole),
            pl.BlockSpec((_K, _D), whole),
        ],
        out_specs=(
            pl.BlockSpec((_BB, _D), row_block),
            pl.BlockSpec((_BB, 1), row_block),
        ),
        out_shape=(
            jax.ShapeDtypeStruct((_B, _D), jnp.float32),
            jax.ShapeDtypeStruct((_B, 1), jnp.int32),
        ),
        compiler_params=pltpu.CompilerParams(
            dimension_semantics=("arbitrary",),
        ),
    )(skills, language_operators, W0, b0[None, :], W1, b1[None, :],
      W2, b2[None, :], codebook)


def _sc_gather(codebook_padded, idx):
    info = plsc.get_sparse_core_info()
    nc, ns = info.num_cores, info.num_subcores
    nw = nc * ns
    b_per_w = _B // nw
    mesh = plsc.VectorSubcoreMesh(core_axis_name="c", subcore_axis_name="s")

    @functools.partial(
        pl.kernel, mesh=mesh,
        out_type=jax.ShapeDtypeStruct((_B, _DP), jnp.float32),
        scratch_types=[
            pltpu.VMEM((b_per_w,), jnp.int32),
            pltpu.VMEM((b_per_w, _DP), jnp.float32),
            pltpu.SemaphoreType.DMA,
        ],
    )
    def gather_kernel(cb_hbm, idx_hbm, out_hbm, idx_v, rows_v, sem):
        wid = lax.axis_index("s") * nc + lax.axis_index("c")
        base = wid * b_per_w
        pltpu.sync_copy(idx_hbm.at[pl.ds(base, b_per_w)], idx_v)
        pltpu.async_copy(cb_hbm.at[idx_v], rows_v, sem).wait()
        pltpu.sync_copy(rows_v, out_hbm.at[pl.ds(base, b_per_w)])

    return gather_kernel(codebook_padded, idx)


def kernel(skills, language_operators, W0, b0, W1, b1, W2, b2, codebook):
    u, idx2d = _tc_stage(skills, language_operators, W0, b0, W1, b1, W2, b2,
                         codebook)
    cb_pad = jnp.pad(codebook, ((0, 0), (0, _DP - _D)))
    q = _sc_gather(cb_pad, idx2d.reshape(_B))[:, :_D]
    return (u, q)


# speedup vs baseline: 1.4071x; 1.0011x over previous
import functools

import jax
import jax.numpy as jnp
from jax import lax
from jax.experimental import pallas as pl
from jax.experimental.pallas import tpu as pltpu
from jax.experimental.pallas import tpu_sc as plsc

_B = 1024
_K = 1024
_D = 64
_BB = 1024
_DP = 128


def _vq_body(s_ref, l_ref, w0_ref, b0_ref, w1_ref, b1_ref, w2_ref, b2_ref,
             cb_ref, u_ref, idx_ref):
    x = jnp.concatenate((s_ref[...], l_ref[...]), axis=1)
    h = jnp.maximum(
        jnp.dot(x, w0_ref[...], preferred_element_type=jnp.float32) + b0_ref[...], 0.0)
    h = jnp.maximum(
        jnp.dot(h, w1_ref[...], preferred_element_type=jnp.float32) + b1_ref[...], 0.0)
    u = jnp.dot(h, w2_ref[...], preferred_element_type=jnp.float32) + b2_ref[...]
    u_ref[...] = u

    u_aug = jnp.concatenate((u * -2.0, jnp.ones((_BB, 1), jnp.float32)), axis=1)
    cb = cb_ref[...]
    c2 = jnp.sum(cb * cb, axis=1, keepdims=True)
    cb_aug = jnp.concatenate((cb, c2), axis=1)
    d = lax.dot_general(u_aug, cb_aug, (((1,), (1,)), ((), ())),
                        preferred_element_type=jnp.float32,
                        precision=lax.Precision.HIGHEST)
    dmin = jnp.min(d, axis=1, keepdims=True)
    iota = lax.broadcasted_iota(jnp.int32, d.shape, 1)
    idx_ref[...] = jnp.min(jnp.where(d == dmin, iota, _K), axis=1,
                           keepdims=True)


def _tc_stage(skills, language_operators, W0, b0, W1, b1, W2, b2, codebook):
    nsteps = _B // _BB
    row_block = lambda i: (i, 0)
    whole = lambda i: (0, 0)
    return pl.pallas_call(
        _vq_body,
        grid=(nsteps,),
        in_specs=[
            pl.BlockSpec((_BB, 128), row_block),
            pl.BlockSpec((_BB, 512), row_block),
            pl.BlockSpec((640, 256), whole),
            pl.BlockSpec((1, 256), whole),
            pl.BlockSpec((256, 256), whole),
            pl.BlockSpec((1, 256), whole),
            pl.BlockSpec((256, _D), whole),
            pl.BlockSpec((1, _D), whole),
            pl.BlockSpec((_K, _D), whole),
        ],
        out_specs=(
            pl.BlockSpec((_BB, _D), row_block),
            pl.BlockSpec((_BB, 1), row_block),
        ),
        out_shape=(
            jax.ShapeDtypeStruct((_B, _D), jnp.float32),
            jax.ShapeDtypeStruct((_B, 1), jnp.int32),
        ),
        compiler_params=pltpu.CompilerParams(
            dimension_semantics=("arbitrary",),
        ),
    )(skills, language_operators, W0, b0[None, :], W1, b1[None, :],
      W2, b2[None, :], codebook)


def _sc_gather(codebook_padded, idx):
    info = plsc.get_sparse_core_info()
    nc, ns = info.num_cores, info.num_subcores
    nw = nc * ns
    b_per_w = _B // nw
    mesh = plsc.VectorSubcoreMesh(core_axis_name="c", subcore_axis_name="s")

    @functools.partial(
        pl.kernel, mesh=mesh,
        out_type=jax.ShapeDtypeStruct((_B, _DP), jnp.float32),
        scratch_types=[
            pltpu.VMEM((b_per_w,), jnp.int32),
            pltpu.VMEM((b_per_w, _DP), jnp.float32),
            pltpu.SemaphoreType.DMA,
        ],
    )
    def gather_kernel(cb_hbm, idx_hbm, out_hbm, idx_v, rows_v, sem):
        wid = lax.axis_index("s") * nc + lax.axis_index("c")
        base = wid * b_per_w
        pltpu.sync_copy(idx_hbm.at[pl.ds(base, b_per_w)], idx_v)
        pltpu.async_copy(cb_hbm.at[idx_v], rows_v, sem).wait()
        pltpu.sync_copy(rows_v, out_hbm.at[pl.ds(base, b_per_w)])

    return gather_kernel(codebook_padded, idx)


def kernel(skills, language_operators, W0, b0, W1, b1, W2, b2, codebook):
    u, idx2d = _tc_stage(skills, language_operators, W0, b0, W1, b1, W2, b2,
                         codebook)
    cb_pad = jnp.pad(codebook, ((0, 0), (0, _DP - _D)))
    q = _sc_gather(cb_pad, idx2d.reshape(_B))[:, :_D]
    return (u, q)


# speedup vs baseline: 2.8504x; 2.0258x over previous
import functools

import jax
import jax.numpy as jnp
from jax import lax
from jax.experimental import pallas as pl
from jax.experimental.pallas import tpu as pltpu
from jax.experimental.pallas import tpu_sc as plsc

_B = 1024
_K = 1024
_D = 64
_BB = 1024
_DP = 128


def _vq_body(s_ref, l_ref, w0_ref, b0_ref, w1_ref, b1_ref, w2_ref, b2_ref,
             cb_ref, u_ref, idx_ref):
    x = jnp.concatenate((s_ref[...], l_ref[...]), axis=1)
    h = jnp.maximum(
        jnp.dot(x, w0_ref[...], preferred_element_type=jnp.float32) + b0_ref[...], 0.0)
    h = jnp.maximum(
        jnp.dot(h, w1_ref[...], preferred_element_type=jnp.float32) + b1_ref[...], 0.0)
    u = jnp.dot(h, w2_ref[...], preferred_element_type=jnp.float32) + b2_ref[...]
    u_ref[...] = u

    u_aug = jnp.concatenate((u * -2.0, jnp.ones((_BB, 1), jnp.float32)), axis=1)
    cb = cb_ref[...]
    c2 = jnp.sum(cb * cb, axis=1, keepdims=True)
    cb_aug = jnp.concatenate((cb, c2), axis=1)
    d = lax.dot_general(u_aug, cb_aug, (((1,), (1,)), ((), ())),
                        preferred_element_type=jnp.float32,
                        precision=lax.Precision.HIGHEST)
    dmin = jnp.min(d, axis=1, keepdims=True)
    iota = lax.broadcasted_iota(jnp.int32, d.shape, 1)
    idx = jnp.min(jnp.where(d == dmin, iota, _K), axis=1, keepdims=True)
    onehot = (lax.broadcasted_iota(jnp.int32, (_BB, _K), 1) == idx).astype(jnp.float32)
    idx_ref[...] = jnp.dot(onehot, cb, preferred_element_type=jnp.float32,
                           precision=lax.Precision.HIGHEST)


def _tc_stage(skills, language_operators, W0, b0, W1, b1, W2, b2, codebook):
    nsteps = _B // _BB
    row_block = lambda i: (i, 0)
    whole = lambda i: (0, 0)
    return pl.pallas_call(
        _vq_body,
        grid=(nsteps,),
        in_specs=[
            pl.BlockSpec((_BB, 128), row_block),
            pl.BlockSpec((_BB, 512), row_block),
            pl.BlockSpec((640, 256), whole),
            pl.BlockSpec((1, 256), whole),
            pl.BlockSpec((256, 256), whole),
            pl.BlockSpec((1, 256), whole),
            pl.BlockSpec((256, _D), whole),
            pl.BlockSpec((1, _D), whole),
            pl.BlockSpec((_K, _D), whole),
        ],
        out_specs=(
            pl.BlockSpec((_BB, _D), row_block),
            pl.BlockSpec((_BB, _D), row_block),
        ),
        out_shape=(
            jax.ShapeDtypeStruct((_B, _D), jnp.float32),
            jax.ShapeDtypeStruct((_B, _D), jnp.float32),
        ),
        compiler_params=pltpu.CompilerParams(
            dimension_semantics=("arbitrary",),
        ),
    )(skills, language_operators, W0, b0[None, :], W1, b1[None, :],
      W2, b2[None, :], codebook)


def _sc_gather(codebook_padded, idx):
    info = plsc.get_sparse_core_info()
    nc, ns = info.num_cores, info.num_subcores
    nw = nc * ns
    b_per_w = _B // nw
    mesh = plsc.VectorSubcoreMesh(core_axis_name="c", subcore_axis_name="s")

    @functools.partial(
        pl.kernel, mesh=mesh,
        out_type=jax.ShapeDtypeStruct((_B, _DP), jnp.float32),
        scratch_types=[
            pltpu.VMEM((b_per_w,), jnp.int32),
            pltpu.VMEM((b_per_w, _DP), jnp.float32),
            pltpu.SemaphoreType.DMA,
        ],
    )
    def gather_kernel(cb_hbm, idx_hbm, out_hbm, idx_v, rows_v, sem):
        wid = lax.axis_index("s") * nc + lax.axis_index("c")
        base = wid * b_per_w
        pltpu.sync_copy(idx_hbm.at[pl.ds(base, b_per_w)], idx_v)
        pltpu.async_copy(cb_hbm.at[idx_v], rows_v, sem).wait()
        pltpu.sync_copy(rows_v, out_hbm.at[pl.ds(base, b_per_w)])

    return gather_kernel(codebook_padded, idx)


def kernel(skills, language_operators, W0, b0, W1, b1, W2, b2, codebook):
    u, q = _tc_stage(skills, language_operators, W0, b0, W1, b1, W2, b2,
                     codebook)
    return (u, q)
